# baseline (speedup 1.0000x reference)
import functools

import jax
import jax.numpy as jnp
from jax import lax
from jax.experimental import pallas as pl
from jax.experimental.pallas import tpu as pltpu
from jax.experimental.pallas import tpu_sc as plsc

NC = 2
NS = 16
CHUNK = 128
NBUF = 4


def _vector_mesh():
    return plsc.VectorSubcoreMesh(core_axis_name="c", subcore_axis_name="s",
                                  num_cores=NC, num_subcores=NS)


def _deg_kernel_body(cpt, rpt,
                     dst_hbm, ones_hbm, zeros_hbm, out_hbm,
                     idx_v, ones_v, acc, sem):
    c = lax.axis_index("c")
    s = lax.axis_index("s")
    base = s * rpt
    pltpu.sync_copy(zeros_hbm, acc.at[pl.ds(base, rpt)])
    pltpu.sync_copy(ones_hbm, ones_v)
    pltpu.sync_copy(dst_hbm.at[s], idx_v)
    plsc.subcore_barrier()

    @pl.loop(0, cpt)
    def _(j):
        pltpu.sync_copy(ones_v, acc.at[idx_v.at[j]], add=True)

    plsc.subcore_barrier()
    pltpu.sync_copy(acc.at[pl.ds(base, rpt)], out_hbm.at[c, pl.ds(base, rpt)])


def _agg_kernel_body(cpt, rpt,
                     hs_hbm, src_hbm, dst_hbm, zeros_hbm, out_hbm,
                     idx_s, idx_d, rows, acc, sems):
    c = lax.axis_index("c")
    s = lax.axis_index("s")
    base = s * rpt
    pltpu.sync_copy(zeros_hbm, acc.at[pl.ds(base, rpt)])
    pltpu.sync_copy(src_hbm.at[c, s], idx_s)
    pltpu.sync_copy(dst_hbm.at[s], idx_d)
    plsc.subcore_barrier()

    nrow = rows.shape[0] // NBUF
    rbuf = [rows.at[pl.ds(b * nrow, nrow)] for b in range(NBUF)]
    for b in range(NBUF):
        pltpu.async_copy(hs_hbm.at[idx_s.at[b]], rbuf[b], sems.at[b])

    @pl.loop(0, cpt, step=NBUF)
    def _(j):
        for b in range(NBUF):
            i = j + b
            pltpu.make_async_copy(hs_hbm.at[idx_s.at[i]], rbuf[b],
                                  sems.at[b]).wait()
            pltpu.sync_copy(rbuf[b], acc.at[idx_d.at[i]], add=True)

            @pl.when(i + NBUF < cpt)
            def _():
                pltpu.async_copy(hs_hbm.at[idx_s.at[i + NBUF]], rbuf[b],
                                 sems.at[b])

    plsc.subcore_barrier()
    pltpu.sync_copy(acc.at[pl.ds(base, rpt)],
                    out_hbm.at[pl.ds(base, rpt), c])


def _dis_hs_body(deg_ref, x_ref, w_ref, dis_ref, hs_ref):
    deg = deg_ref[0, :, 0:1] + 1.0
    dis = lax.rsqrt(deg)
    xw = jnp.dot(x_ref[...], w_ref[...], preferred_element_type=jnp.float32)
    disb = jnp.broadcast_to(dis, xw.shape)
    dis_ref[...] = disb
    hs_ref[...] = disb * xw


def _mid_body(agg_ref, hs_ref, dis_ref, b_ref, w_ref, o_ref):
    pre = dis_ref[...] * (agg_ref[...] + hs_ref[...]) + b_ref[...]
    act = jnp.maximum(pre, 0.0)
    xw = jnp.dot(act, w_ref[...], preferred_element_type=jnp.float32)
    o_ref[...] = dis_ref[...] * xw


def _fin_body(agg_ref, hs_ref, dis_ref, b_ref, o_ref):
    o_ref[...] = dis_ref[...] * (agg_ref[...] + hs_ref[...]) + b_ref[...]


def kernel(x, edge_index, W1, b1, W2, b2, W3, b3, W4, b4):
    n, din = x.shape
    d = W1.shape[1]
    h = d // 2
    e = edge_index.shape[1]

    cpt = -(-e // (NS * CHUNK))
    cpt = -(-cpt // NBUF) * NBUF
    e_pad = NS * cpt * CHUNK
    rpt = -(-(n + 1) // NS)
    rpt = -(-rpt // 8) * 8
    npad = NS * rpt

    src = edge_index[0].astype(jnp.int32)
    dst = edge_index[1].astype(jnp.int32)
    pad_iota = jnp.arange(e_pad - e, dtype=jnp.int32)
    src = jnp.concatenate([src, pad_iota % n])
    dst = jnp.concatenate([dst, n + pad_iota % (npad - n)])
    src2 = jnp.stack([2 * src, 2 * src + 1]).reshape(NC, NS, cpt, CHUNK)
    dst3 = dst.reshape(NS, cpt, CHUNK)

    ones16 = jnp.ones((CHUNK, 16), jnp.float32)
    zeros16 = jnp.zeros((rpt, 16), jnp.float32)
    zerosh = jnp.zeros((rpt, h), jnp.float32)

    mesh = _vector_mesh()

    deg_k = pl.kernel(
        functools.partial(_deg_kernel_body, cpt, rpt),
        out_type=jax.ShapeDtypeStruct((NC, npad, 16), jnp.float32),
        mesh=mesh,
        compiler_params=pltpu.CompilerParams(use_tc_tiling_on_sc=False),
        scratch_types=[
            pltpu.VMEM((cpt, CHUNK), jnp.int32),
            pltpu.VMEM((CHUNK, 16), jnp.float32),
            pltpu.VMEM_SHARED((npad, 16), jnp.float32),
            pltpu.SemaphoreType.DMA,
        ],
    )

    agg_k = pl.kernel(
        functools.partial(_agg_kernel_body, cpt, rpt),
        out_type=jax.ShapeDtypeStruct((npad, NC, h), jnp.float32),
        mesh=mesh,
        compiler_params=pltpu.CompilerParams(use_tc_tiling_on_sc=False),
        scratch_types=[
            pltpu.VMEM((cpt, CHUNK), jnp.int32),
            pltpu.VMEM((cpt, CHUNK), jnp.int32),
            pltpu.VMEM((NBUF * CHUNK, h), jnp.float32),
            pltpu.VMEM_SHARED((npad, h), jnp.float32),
            pltpu.SemaphoreType.DMA((NBUF,)),
        ],
    )

    br = 2000
    grid = (n // br,)
    row_spec = pl.BlockSpec((br, d), lambda i: (i, 0))

    dis_hs = pl.pallas_call(
        _dis_hs_body,
        grid=grid,
        in_specs=[pl.BlockSpec((NC, br, 16), lambda i: (0, i, 0)),
                  pl.BlockSpec((br, din), lambda i: (i, 0)),
                  pl.BlockSpec((din, d), lambda i: (0, 0))],
        out_specs=[row_spec, row_spec],
        out_shape=[jax.ShapeDtypeStruct((n, d), jnp.float32),
                   jax.ShapeDtypeStruct((n, d), jnp.float32)],
    )

    mid = pl.pallas_call(
        _mid_body,
        grid=grid,
        in_specs=[row_spec, row_spec, row_spec,
                  pl.BlockSpec((1, d), lambda i: (0, 0)),
                  pl.BlockSpec((d, d), lambda i: (0, 0))],
        out_specs=row_spec,
        out_shape=jax.ShapeDtypeStruct((n, d), jnp.float32),
    )

    fin = pl.pallas_call(
        _fin_body,
        grid=grid,
        in_specs=[row_spec, row_spec, row_spec,
                  pl.BlockSpec((1, d), lambda i: (0, 0))],
        out_specs=row_spec,
        out_shape=jax.ShapeDtypeStruct((n, d), jnp.float32),
    )

    def agg(hs_full):
        hs_lin = jnp.reshape(hs_full, (2 * n, h))
        agg_p = agg_k(hs_lin, src2, dst3, zerosh)
        return jnp.reshape(agg_p, (npad, d))

    deg_p = deg_k(dst3, ones16, zeros16)
    dis, hs = dis_hs(deg_p, x, W1)

    agg1 = agg(hs)
    hs = mid(agg1, hs, dis, b1.reshape(1, d), W2)
    agg2 = agg(hs)
    hs = mid(agg2, hs, dis, b2.reshape(1, d), W3)
    agg3 = agg(hs)
    hs = mid(agg3, hs, dis, b3.reshape(1, d), W4)
    agg4 = agg(hs)
    return fin(agg4, hs, dis, b4.reshape(1, d))


# speedup vs baseline: 1.1987x; 1.1987x over previous
import functools

import jax
import jax.numpy as jnp
from jax import lax
from jax.experimental import pallas as pl
from jax.experimental.pallas import tpu as pltpu
from jax.experimental.pallas import tpu_sc as plsc

NC = 2
NS = 16
CHUNK = 128
NBUF = 4


def _vector_mesh():
    return plsc.VectorSubcoreMesh(core_axis_name="c", subcore_axis_name="s",
                                  num_cores=NC, num_subcores=NS)


def _deg_kernel_body(cpt, rpt,
                     dst_hbm, ones_hbm, zeros_hbm, out_hbm,
                     idx_v, ones_v, acc, sem):
    c = lax.axis_index("c")
    s = lax.axis_index("s")
    base = s * rpt
    pltpu.sync_copy(zeros_hbm, acc.at[pl.ds(base, rpt)])
    pltpu.sync_copy(ones_hbm, ones_v)
    pltpu.sync_copy(dst_hbm.at[s], idx_v)
    plsc.subcore_barrier()

    @pl.loop(0, cpt)
    def _(j):
        pltpu.sync_copy(ones_v, acc.at[idx_v.at[j]], add=True)

    plsc.subcore_barrier()
    pltpu.sync_copy(acc.at[pl.ds(base, rpt)], out_hbm.at[c, pl.ds(base, rpt)])


def _agg_kernel_body(cpt, rpt, h,
                     hs_hbm, src_hbm, dst_hbm, zeros_hbm, out_hbm,
                     idx_s, idx_d, rows, acc, sems):
    c = lax.axis_index("c")
    s = lax.axis_index("s")
    base = s * rpt
    pltpu.sync_copy(zeros_hbm, acc.at[pl.ds(base, rpt)])
    pltpu.sync_copy(src_hbm.at[s], idx_s)
    pltpu.sync_copy(dst_hbm.at[s], idx_d)
    plsc.subcore_barrier()
    table = hs_hbm.at[c]

    nrow = rows.shape[0] // NBUF
    rbuf = [rows.at[pl.ds(b * nrow, nrow)] for b in range(NBUF)]
    for b in range(NBUF):
        pltpu.async_copy(table.at[idx_s.at[b]], rbuf[b], sems.at[b])

    @pl.loop(0, cpt, step=NBUF)
    def _(j):
        for b in range(NBUF):
            i = j + b
            pltpu.make_async_copy(table.at[idx_s.at[i]], rbuf[b],
                                  sems.at[b]).wait()
            pltpu.sync_copy(rbuf[b], acc.at[idx_d.at[i]], add=True)

            @pl.when(i + NBUF < cpt)
            def _():
                pltpu.async_copy(table.at[idx_s.at[i + NBUF]], rbuf[b],
                                 sems.at[b])

    plsc.subcore_barrier()
    pltpu.sync_copy(acc.at[pl.ds(base, rpt)],
                    out_hbm.at[pl.ds(base, rpt), pl.ds(c * h, h)])


def _split(o_ref, full, h):
    o_ref[0, :, :] = full[:, :h]
    o_ref[1, :, :] = full[:, h:]


def _hs_full(hs_ref):
    return jnp.concatenate([hs_ref[0], hs_ref[1]], axis=-1)


def _dis_hs_body(deg_ref, x_ref, w_ref, dis_ref, hs_ref):
    deg = deg_ref[0, :, 0:1] + 1.0
    dis = lax.rsqrt(deg)
    xw = jnp.dot(x_ref[...], w_ref[...], preferred_element_type=jnp.float32)
    disb = jnp.broadcast_to(dis, xw.shape)
    dis_ref[...] = disb
    _split(hs_ref, disb * xw, xw.shape[1] // 2)


def _mid_body(agg_ref, hs_ref, dis_ref, b_ref, w_ref, o_ref):
    pre = dis_ref[...] * (agg_ref[...] + _hs_full(hs_ref)) + b_ref[...]
    act = jnp.maximum(pre, 0.0)
    xw = jnp.dot(act, w_ref[...], preferred_element_type=jnp.float32)
    _split(o_ref, dis_ref[...] * xw, xw.shape[1] // 2)


def _fin_body(agg_ref, hs_ref, dis_ref, b_ref, o_ref):
    o_ref[...] = dis_ref[...] * (agg_ref[...] + _hs_full(hs_ref)) + b_ref[...]


def kernel(x, edge_index, W1, b1, W2, b2, W3, b3, W4, b4):
    n, din = x.shape
    d = W1.shape[1]
    h = d // 2
    e = edge_index.shape[1]

    cpt = -(-e // (NS * CHUNK))
    cpt = -(-cpt // NBUF) * NBUF
    e_pad = NS * cpt * CHUNK
    rpt = -(-(n + 1) // NS)
    rpt = -(-rpt // 8) * 8
    npad = NS * rpt

    src = edge_index[0].astype(jnp.int32)
    dst = edge_index[1].astype(jnp.int32)
    pad_iota = jnp.arange(e_pad - e, dtype=jnp.int32)
    src = jnp.concatenate([src, pad_iota % n])
    dst = jnp.concatenate([dst, n + pad_iota % (npad - n)])
    src3 = src.reshape(NS, cpt, CHUNK)
    dst3 = dst.reshape(NS, cpt, CHUNK)

    ones16 = jnp.ones((CHUNK, 16), jnp.float32)
    zeros16 = jnp.zeros((rpt, 16), jnp.float32)
    zerosh = jnp.zeros((rpt, h), jnp.float32)

    mesh = _vector_mesh()

    deg_k = pl.kernel(
        functools.partial(_deg_kernel_body, cpt, rpt),
        out_type=jax.ShapeDtypeStruct((NC, npad, 16), jnp.float32),
        mesh=mesh,
        compiler_params=pltpu.CompilerParams(use_tc_tiling_on_sc=False),
        scratch_types=[
            pltpu.VMEM((cpt, CHUNK), jnp.int32),
            pltpu.VMEM((CHUNK, 16), jnp.float32),
            pltpu.VMEM_SHARED((npad, 16), jnp.float32),
            pltpu.SemaphoreType.DMA,
        ],
    )

    agg_k = pl.kernel(
        functools.partial(_agg_kernel_body, cpt, rpt, h),
        out_type=jax.ShapeDtypeStruct((npad, d), jnp.float32),
        mesh=mesh,
        compiler_params=pltpu.CompilerParams(use_tc_tiling_on_sc=False),
        scratch_types=[
            pltpu.VMEM((cpt, CHUNK), jnp.int32),
            pltpu.VMEM((cpt, CHUNK), jnp.int32),
            pltpu.VMEM((NBUF * CHUNK, h), jnp.float32),
            pltpu.VMEM_SHARED((npad, h), jnp.float32),
            pltpu.SemaphoreType.DMA((NBUF,)),
        ],
    )

    br = 2000
    grid = (n // br,)
    row_spec = pl.BlockSpec((br, d), lambda i: (i, 0))
    hs_spec = pl.BlockSpec((NC, br, h), lambda i: (0, i, 0))
    hs_shape = jax.ShapeDtypeStruct((NC, n, h), jnp.float32)

    dis_hs = pl.pallas_call(
        _dis_hs_body,
        grid=grid,
        in_specs=[pl.BlockSpec((NC, br, 16), lambda i: (0, i, 0)),
                  pl.BlockSpec((br, din), lambda i: (i, 0)),
                  pl.BlockSpec((din, d), lambda i: (0, 0))],
        out_specs=[row_spec, hs_spec],
        out_shape=[jax.ShapeDtypeStruct((n, d), jnp.float32), hs_shape],
    )

    mid = pl.pallas_call(
        _mid_body,
        grid=grid,
        in_specs=[row_spec, hs_spec, row_spec,
                  pl.BlockSpec((1, d), lambda i: (0, 0)),
                  pl.BlockSpec((d, d), lambda i: (0, 0))],
        out_specs=hs_spec,
        out_shape=hs_shape,
    )

    fin = pl.pallas_call(
        _fin_body,
        grid=grid,
        in_specs=[row_spec, hs_spec, row_spec,
                  pl.BlockSpec((1, d), lambda i: (0, 0))],
        out_specs=row_spec,
        out_shape=jax.ShapeDtypeStruct((n, d), jnp.float32),
    )

    def agg(hs_split):
        return agg_k(hs_split, src3, dst3, zerosh)

    deg_p = deg_k(dst3, ones16, zeros16)
    dis, hs = dis_hs(deg_p, x, W1)

    agg1 = agg(hs)
    hs = mid(agg1, hs, dis, b1.reshape(1, d), W2)
    agg2 = agg(hs)
    hs = mid(agg2, hs, dis, b2.reshape(1, d), W3)
    agg3 = agg(hs)
    hs = mid(agg3, hs, dis, b3.reshape(1, d), W4)
    agg4 = agg(hs)
    return fin(agg4, hs, dis, b4.reshape(1, d))


# speedup vs baseline: 1.2073x; 1.0071x over previous
import functools

import jax
import jax.numpy as jnp
from jax import lax
from jax.experimental import pallas as pl
from jax.experimental.pallas import tpu as pltpu
from jax.experimental.pallas import tpu_sc as plsc

NC = 2
NS = 16
CHUNK = 128
NBUF = 4


def _vector_mesh():
    return plsc.VectorSubcoreMesh(core_axis_name="c", subcore_axis_name="s",
                                  num_cores=NC, num_subcores=NS)


def _deg_kernel_body(cpt, rpt,
                     dst_hbm, ones_hbm, zeros_hbm, out_hbm,
                     idx_v, ones_v, acc, sem):
    c = lax.axis_index("c")
    s = lax.axis_index("s")
    base = s * rpt
    pltpu.sync_copy(zeros_hbm.at[s], acc.at[pl.ds(base, rpt)])
    pltpu.sync_copy(ones_hbm, ones_v)
    pltpu.sync_copy(dst_hbm.at[s], idx_v)
    plsc.subcore_barrier()

    @pl.loop(0, cpt)
    def _(j):
        pltpu.sync_copy(ones_v, acc.at[idx_v.at[j]], add=True)

    plsc.subcore_barrier()
    pltpu.sync_copy(acc.at[pl.ds(base, rpt)], out_hbm.at[c, pl.ds(base, rpt)])


def _agg_kernel_body(cpt, rpt, h,
                     hs_hbm, src_hbm, dst_hbm, zeros_hbm, out_hbm,
                     idx_s, idx_d, rows, acc, sems):
    c = lax.axis_index("c")
    s = lax.axis_index("s")
    base = s * rpt
    pltpu.sync_copy(zeros_hbm.at[s], acc.at[pl.ds(base, rpt)])
    pltpu.sync_copy(src_hbm.at[s], idx_s)
    pltpu.sync_copy(dst_hbm.at[s], idx_d)
    plsc.subcore_barrier()
    table = hs_hbm.at[c]

    nrow = rows.shape[0] // NBUF
    rbuf = [rows.at[pl.ds(b * nrow, nrow)] for b in range(NBUF)]
    for b in range(NBUF):
        pltpu.async_copy(table.at[idx_s.at[b]], rbuf[b], sems.at[b])

    @pl.loop(0, cpt, step=NBUF)
    def _(j):
        for b in range(NBUF):
            i = j + b
            pltpu.make_async_copy(table.at[idx_s.at[i]], rbuf[b],
                                  sems.at[b]).wait()
            pltpu.sync_copy(rbuf[b], acc.at[idx_d.at[i]], add=True)

            @pl.when(i + NBUF < cpt)
            def _():
                pltpu.async_copy(table.at[idx_s.at[i + NBUF]], rbuf[b],
                                 sems.at[b])

    plsc.subcore_barrier()
    pltpu.sync_copy(acc.at[pl.ds(base, rpt)],
                    out_hbm.at[pl.ds(base, rpt), pl.ds(c * h, h)])


def _split(o_ref, full, h):
    o_ref[0, :, :] = full[:, :h]
    o_ref[1, :, :] = full[:, h:]


def _hs_full(hs_ref):
    return jnp.concatenate([hs_ref[0], hs_ref[1]], axis=-1)


def _dis_hs_body(deg_ref, x_ref, w_ref, dis_ref, hs_ref):
    deg = deg_ref[0, :, 0:1] + 1.0
    dis = lax.rsqrt(deg)
    xw = jnp.dot(x_ref[...], w_ref[...], preferred_element_type=jnp.float32)
    disb = jnp.broadcast_to(dis, xw.shape)
    dis_ref[...] = disb
    _split(hs_ref, disb * xw, xw.shape[1] // 2)


def _mid_body(agg_ref, hs_ref, dis_ref, b_ref, w_ref, o_ref):
    pre = dis_ref[...] * (agg_ref[...] + _hs_full(hs_ref)) + b_ref[...]
    act = jnp.maximum(pre, 0.0)
    xw = jnp.dot(act, w_ref[...], preferred_element_type=jnp.float32)
    _split(o_ref, dis_ref[...] * xw, xw.shape[1] // 2)


def _fin_body(agg_ref, hs_ref, dis_ref, b_ref, o_ref):
    o_ref[...] = dis_ref[...] * (agg_ref[...] + _hs_full(hs_ref)) + b_ref[...]


def kernel(x, edge_index, W1, b1, W2, b2, W3, b3, W4, b4):
    n, din = x.shape
    d = W1.shape[1]
    h = d // 2
    e = edge_index.shape[1]

    cpt = -(-e // (NS * CHUNK))
    cpt = -(-cpt // NBUF) * NBUF
    e_pad = NS * cpt * CHUNK
    rpt = -(-(n + 1) // NS)
    rpt = -(-rpt // 8) * 8
    npad = NS * rpt

    src = edge_index[0].astype(jnp.int32)
    dst = edge_index[1].astype(jnp.int32)
    pad_iota = jnp.arange(e_pad - e, dtype=jnp.int32)
    src = jnp.concatenate([src, pad_iota % n])
    dst = jnp.concatenate([dst, n + pad_iota % (npad - n)])
    src3 = src.reshape(NS, cpt, CHUNK)
    dst3 = dst.reshape(NS, cpt, CHUNK)

    ones16 = jnp.ones((CHUNK, 16), jnp.float32)
    zeros16 = jnp.zeros((NS, rpt, 16), jnp.float32)
    zerosh = jnp.zeros((NS, rpt, h), jnp.float32)

    mesh = _vector_mesh()

    deg_k = pl.kernel(
        functools.partial(_deg_kernel_body, cpt, rpt),
        out_type=jax.ShapeDtypeStruct((NC, npad, 16), jnp.float32),
        mesh=mesh,
        compiler_params=pltpu.CompilerParams(use_tc_tiling_on_sc=False),
        scratch_types=[
            pltpu.VMEM((cpt, CHUNK), jnp.int32),
            pltpu.VMEM((CHUNK, 16), jnp.float32),
            pltpu.VMEM_SHARED((npad, 16), jnp.float32),
            pltpu.SemaphoreType.DMA,
        ],
    )

    agg_k = pl.kernel(
        functools.partial(_agg_kernel_body, cpt, rpt, h),
        out_type=jax.ShapeDtypeStruct((npad, d), jnp.float32),
        mesh=mesh,
        compiler_params=pltpu.CompilerParams(use_tc_tiling_on_sc=False),
        scratch_types=[
            pltpu.VMEM((cpt, CHUNK), jnp.int32),
            pltpu.VMEM((cpt, CHUNK), jnp.int32),
            pltpu.VMEM((NBUF * CHUNK, h), jnp.float32),
            pltpu.VMEM_SHARED((npad, h), jnp.float32),
            pltpu.SemaphoreType.DMA((NBUF,)),
        ],
    )

    br = 2000
    grid = (n // br,)
    row_spec = pl.BlockSpec((br, d), lambda i: (i, 0))
    hs_spec = pl.BlockSpec((NC, br, h), lambda i: (0, i, 0))
    hs_shape = jax.ShapeDtypeStruct((NC, n, h), jnp.float32)

    dis_hs = pl.pallas_call(
        _dis_hs_body,
        grid=grid,
        in_specs=[pl.BlockSpec((NC, br, 16), lambda i: (0, i, 0)),
                  pl.BlockSpec((br, din), lambda i: (i, 0)),
                  pl.BlockSpec((din, d), lambda i: (0, 0))],
        out_specs=[row_spec, hs_spec],
        out_shape=[jax.ShapeDtypeStruct((n, d), jnp.float32), hs_shape],
    )

    mid = pl.pallas_call(
        _mid_body,
        grid=grid,
        in_specs=[row_spec, hs_spec, row_spec,
                  pl.BlockSpec((1, d), lambda i: (0, 0)),
                  pl.BlockSpec((d, d), lambda i: (0, 0))],
        out_specs=hs_spec,
        out_shape=hs_shape,
    )

    fin = pl.pallas_call(
        _fin_body,
        grid=grid,
        in_specs=[row_spec, hs_spec, row_spec,
                  pl.BlockSpec((1, d), lambda i: (0, 0))],
        out_specs=row_spec,
        out_shape=jax.ShapeDtypeStruct((n, d), jnp.float32),
    )

    def agg(hs_split):
        return agg_k(hs_split, src3, dst3, zerosh)

    deg_p = deg_k(dst3, ones16, zeros16)
    dis, hs = dis_hs(deg_p, x, W1)

    agg1 = agg(hs)
    hs = mid(agg1, hs, dis, b1.reshape(1, d), W2)
    agg2 = agg(hs)
    hs = mid(agg2, hs, dis, b2.reshape(1, d), W3)
    agg3 = agg(hs)
    hs = mid(agg3, hs, dis, b3.reshape(1, d), W4)
    agg4 = agg(hs)
    return fin(agg4, hs, dis, b4.reshape(1, d))
